# baseline (speedup 1.0000x reference)
import functools

import jax
import jax.numpy as jnp
from jax import lax
from jax.experimental import pallas as pl
from jax.experimental.pallas import tpu as pltpu
from jax.experimental.pallas import tpu_sc as plsc

N = 10000
E = 320000
DIN = 128
G = 64

NC = 2
NS = 16
RPT = 624
RPT_LAST = N - (NS - 1) * RPT
CHUNK = 80
EPW = E // (NC * NS)
NCHUNKS = EPW // CHUNK
EPC = E // NC
DW = 16


def _mesh():
    return plsc.VectorSubcoreMesh(core_axis_name="c", subcore_axis_name="s")


def _copy_tile_rows(src_ref, dst_ref, s):
    r0 = s * RPT

    @pl.when(s < NS - 1)
    def _():
        pltpu.sync_copy(src_ref.at[pl.ds(r0, RPT)], dst_ref.at[pl.ds(r0, RPT)])

    @pl.when(s == NS - 1)
    def _():
        pltpu.sync_copy(src_ref.at[pl.ds((NS - 1) * RPT, RPT_LAST)],
                        dst_ref.at[pl.ds((NS - 1) * RPT, RPT_LAST)])


def _make_agg(F):
    NBUF = 3 if F == 128 else 4
    NGRP = NCHUNKS // NBUF
    REM = NCHUNKS - NGRP * NBUF

    @functools.partial(
        pl.kernel,
        out_type=jax.ShapeDtypeStruct((NC, N, F), jnp.float32),
        mesh=_mesh(),
        compiler_params=pltpu.CompilerParams(use_tc_tiling_on_sc=False),
        scratch_types=[
            pltpu.VMEM((NCHUNKS, CHUNK), jnp.int32),
            pltpu.VMEM((NCHUNKS, CHUNK), jnp.int32),
            [pltpu.VMEM((CHUNK, F), jnp.float32)] * NBUF,
            pltpu.VMEM_SHARED((N, F), jnp.float32),
            [pltpu.SemaphoreType.DMA] * NBUF,
            [pltpu.SemaphoreType.DMA] * NBUF,
        ],
    )
    def agg(table_hbm, src_hbm, dst_hbm, zeros_hbm, out_hbm,
            sidx_v, didx_v, rows, acc_sh, gsem, ssem):
        c = lax.axis_index("c")
        s = lax.axis_index("s")
        chunk0 = c * (EPC // CHUNK) + s * NCHUNKS
        pltpu.sync_copy(src_hbm.at[pl.ds(chunk0, NCHUNKS)], sidx_v)
        pltpu.sync_copy(dst_hbm.at[pl.ds(chunk0, NCHUNKS)], didx_v)
        _copy_tile_rows(zeros_hbm, acc_sh, s)
        plsc.subcore_barrier()

        for b in range(NBUF):
            pltpu.async_copy(table_hbm.at[sidx_v.at[b]], rows[b], gsem[b])

        def body(g, carry):
            i0 = NBUF * g
            for b in range(NBUF):
                pltpu.make_async_copy(
                    table_hbm.at[sidx_v.at[i0 + b]], rows[b], gsem[b]).wait()
                pltpu.async_copy(rows[b], acc_sh.at[didx_v.at[i0 + b]],
                                 ssem[b], add=True)
            for b in range(NBUF):
                nxt = i0 + b + NBUF

                @pl.when(nxt < NCHUNKS)
                def _():
                    pltpu.make_async_copy(
                        rows[b], acc_sh.at[didx_v.at[i0 + b]], ssem[b]).wait()
                    pltpu.async_copy(
                        table_hbm.at[sidx_v.at[nxt]], rows[b], gsem[b])
            return carry

        lax.fori_loop(0, NGRP, body, 0)
        for r in range(NGRP * NBUF, NCHUNKS):
            b = r % NBUF
            pltpu.make_async_copy(
                table_hbm.at[sidx_v.at[r]], rows[b], gsem[b]).wait()
            pltpu.async_copy(rows[b], acc_sh.at[didx_v.at[r]],
                             ssem[b], add=True)
            pltpu.make_async_copy(
                rows[b], acc_sh.at[didx_v.at[r]], ssem[b]).wait()
        for b in range(REM, NBUF):
            k = NBUF * (NGRP - 1) + b
            pltpu.make_async_copy(
                rows[b], acc_sh.at[didx_v.at[k]], ssem[b]).wait()
        plsc.subcore_barrier()
        _copy_tile_rows(acc_sh, out_hbm.at[c], s)

    return agg


@functools.cache
def _get_agg(F):
    return _make_agg(F)


_DEG_LAG = 16


def _deg_body(dst_hbm, zeros_hbm, ones_hbm, out_hbm, didx_v, ones_v, acc_sh,
              ssem):
    c = lax.axis_index("c")
    s = lax.axis_index("s")
    chunk0 = c * (EPC // CHUNK) + s * NCHUNKS
    pltpu.sync_copy(ones_hbm, ones_v)
    pltpu.sync_copy(dst_hbm.at[pl.ds(chunk0, NCHUNKS)], didx_v)
    _copy_tile_rows(zeros_hbm, acc_sh, s)
    plsc.subcore_barrier()

    def body(i, carry):
        pltpu.async_copy(ones_v, acc_sh.at[didx_v.at[i]], ssem, add=True)

        @pl.when(i >= _DEG_LAG)
        def _():
            pltpu.make_async_copy(
                ones_v, acc_sh.at[didx_v.at[i]], ssem).wait()
        return carry

    lax.fori_loop(0, NCHUNKS, body, 0)

    def drain(i, carry):
        pltpu.make_async_copy(ones_v, acc_sh.at[didx_v.at[0]], ssem).wait()
        return carry

    lax.fori_loop(0, _DEG_LAG, drain, 0)
    plsc.subcore_barrier()
    _copy_tile_rows(acc_sh, out_hbm.at[c], s)


@functools.cache
def _get_deg():
    return functools.partial(
        pl.kernel,
        out_type=jax.ShapeDtypeStruct((NC, N, DW), jnp.float32),
        mesh=_mesh(),
        compiler_params=pltpu.CompilerParams(use_tc_tiling_on_sc=False),
        scratch_types=[
            pltpu.VMEM((NCHUNKS, CHUNK), jnp.int32),
            pltpu.VMEM((CHUNK, DW), jnp.float32),
            pltpu.VMEM_SHARED((N, DW), jnp.float32),
            pltpu.SemaphoreType.DMA,
        ],
    )(_deg_body)


def _dinv_xp_body(degp_ref, x_ref, dinv_ref, xp_ref):
    dp = degp_ref[...]
    deg = dp[0, :, 0:1] + dp[1, :, 0:1] + 1.0
    dinv = lax.rsqrt(deg)
    dinv_ref[...] = dinv
    xp_ref[...] = x_ref[...] * dinv


_dinv_xp = pl.pallas_call(
    _dinv_xp_body,
    out_shape=[
        jax.ShapeDtypeStruct((N, 1), jnp.float32),
        jax.ShapeDtypeStruct((N, DIN), jnp.float32),
    ],
)


def _l1_body(s_ref, xp_ref, dinv_ref, W1_ref, b1_ref, W2_ref, out_ref):
    sp = s_ref[...]
    dinv = dinv_ref[...]
    agg = (sp[0] + sp[1] + xp_ref[...]) * dinv
    h1 = jnp.maximum(
        jnp.dot(agg, W1_ref[...], preferred_element_type=jnp.float32)
        + b1_ref[...], 0.0)
    out_ref[...] = jnp.dot(
        h1, W2_ref[...], preferred_element_type=jnp.float32) * dinv


_l1 = pl.pallas_call(
    _l1_body,
    out_shape=jax.ShapeDtypeStruct((N, 128), jnp.float32),
)


def _l2_body(s_ref, tp2_ref, dinv_ref, b2_ref, W3_ref, out_ref):
    sp = s_ref[...]
    dinv = dinv_ref[...]
    h2 = jnp.maximum((sp[0] + sp[1] + tp2_ref[...]) * dinv + b2_ref[...], 0.0)
    out_ref[...] = jnp.dot(
        h2, W3_ref[...], preferred_element_type=jnp.float32) * dinv


_l2 = pl.pallas_call(
    _l2_body,
    out_shape=jax.ShapeDtypeStruct((N, 64), jnp.float32),
)


def _l3_body(s_ref, tp3_ref, dinv_ref, b3_ref, out_ref):
    sp = s_ref[...]
    out_ref[...] = jnp.maximum(
        (sp[0] + sp[1] + tp3_ref[...]) * dinv_ref[...] + b3_ref[...], 0.0)


_l3 = pl.pallas_call(
    _l3_body,
    out_shape=jax.ShapeDtypeStruct((N, 64), jnp.float32),
)


def _final_body(h3_ref, batch_ref,
                fW1_ref, fb1_ref, fW2_ref, fb2_ref, fW3_ref, fb3_ref,
                out_ref):
    h3 = h3_ref[...]
    bid = batch_ref[...]
    neg = jnp.float32(-jnp.inf)

    cm = h3
    stride = 1
    while stride < N:
        idsh = jnp.concatenate(
            [jnp.full((stride, 1), -1, jnp.int32), bid[:-stride]], axis=0)
        vsh = jnp.concatenate(
            [jnp.zeros((stride, 64), jnp.float32), cm[:-stride]], axis=0)
        cm = jnp.where(idsh == bid, jnp.maximum(cm, vsh), cm)
        stride *= 2

    nid = jnp.concatenate(
        [bid[1:], jnp.full((1, 1), -1, jnp.int32)], axis=0)
    is_last = nid != bid
    iota_g = lax.broadcasted_iota(jnp.int32, (1, G), 1)
    onehot = jnp.where((bid == iota_g) & is_last, 1.0, 0.0)
    cmz = jnp.where(is_last, cm, 0.0)
    p = lax.dot_general(onehot, cmz, (((0,), (0,)), ((), ())),
                        preferred_element_type=jnp.float32)
    cnt = lax.dot_general(onehot, jnp.ones((N, 1), jnp.float32),
                          (((0,), (0,)), ((), ())),
                          preferred_element_type=jnp.float32)
    p = jnp.where(cnt > 0.0, p, neg)
    z = jnp.maximum(
        jnp.dot(p, fW1_ref[...], preferred_element_type=jnp.float32)
        + fb1_ref[...], 0.0)
    z = jnp.maximum(
        jnp.dot(z, fW2_ref[...], preferred_element_type=jnp.float32)
        + fb2_ref[...], 0.0)
    z = jnp.dot(z, fW3_ref[...], preferred_element_type=jnp.float32) \
        + fb3_ref[...]
    zm = jnp.max(z, axis=1, keepdims=True)
    e = z - zm
    out_ref[...] = e - jnp.log(jnp.sum(jnp.exp(e), axis=1, keepdims=True))


_final = pl.pallas_call(
    _final_body,
    out_shape=jax.ShapeDtypeStruct((G, 10), jnp.float32),
)


def kernel(x, edge_index, batch,
           W1, b1, W2, b2, W3, b3,
           fW1, fb1, fW2, fb2, fW3, fb3):
    src = edge_index[0]
    dst = edge_index[1]
    dst2d = dst.reshape(E // CHUNK, CHUNK)
    src2d = lax.optimization_barrier(src).reshape(E // CHUNK, CHUNK)
    zeros128 = jnp.zeros((N, 128), jnp.float32)
    zeros64 = jnp.zeros((N, 64), jnp.float32)
    zeros16 = jnp.zeros((N, DW), jnp.float32)
    ones16 = jnp.ones((CHUNK, DW), jnp.float32)

    degp = _get_deg()(dst2d, zeros16, ones16)
    dinv, xp = _dinv_xp(degp, x.astype(jnp.float32))

    s1 = _get_agg(128)(xp, src2d, dst2d, zeros128)
    tp2 = _l1(s1, xp, dinv, W1, b1.reshape(1, -1), W2)

    s2 = _get_agg(128)(tp2, src2d, dst2d, zeros128)
    tp3 = _l2(s2, tp2, dinv, b2.reshape(1, -1), W3)

    s3 = _get_agg(64)(tp3, src2d, dst2d, zeros64)
    h3 = _l3(s3, tp3, dinv, b3.reshape(1, -1))
    out = _final(h3, batch.reshape(-1, 1),
                 fW1, fb1.reshape(1, -1), fW2, fb2.reshape(1, -1),
                 fW3, fb3.reshape(1, -1))
    return out


# speedup vs baseline: 1.0514x; 1.0514x over previous
import functools

import jax
import jax.numpy as jnp
from jax import lax
from jax.experimental import pallas as pl
from jax.experimental.pallas import tpu as pltpu
from jax.experimental.pallas import tpu_sc as plsc

N = 10000
E = 320000
DIN = 128
G = 64

NC = 2
NS = 16
RPT = 624
RPT_LAST = N - (NS - 1) * RPT
CHUNK = 80
EPW = E // (NC * NS)
NCHUNKS = EPW // CHUNK
EPC = E // NC
DW = 16


def _mesh():
    return plsc.VectorSubcoreMesh(core_axis_name="c", subcore_axis_name="s")


def _copy_tile_rows(src_ref, dst_ref, s):
    r0 = s * RPT

    @pl.when(s < NS - 1)
    def _():
        pltpu.sync_copy(src_ref.at[pl.ds(r0, RPT)], dst_ref.at[pl.ds(r0, RPT)])

    @pl.when(s == NS - 1)
    def _():
        pltpu.sync_copy(src_ref.at[pl.ds((NS - 1) * RPT, RPT_LAST)],
                        dst_ref.at[pl.ds((NS - 1) * RPT, RPT_LAST)])


def _make_agg(F):
    NBUF = 3 if F == 128 else 4
    NGRP = NCHUNKS // NBUF
    REM = NCHUNKS - NGRP * NBUF

    @functools.partial(
        pl.kernel,
        out_type=jax.ShapeDtypeStruct((NC, N, F), jnp.float32),
        mesh=_mesh(),
        compiler_params=pltpu.CompilerParams(use_tc_tiling_on_sc=False),
        scratch_types=[
            pltpu.VMEM((NCHUNKS, CHUNK), jnp.int32),
            pltpu.VMEM((NCHUNKS, CHUNK), jnp.int32),
            [pltpu.VMEM((CHUNK, F), jnp.float32)] * NBUF,
            pltpu.VMEM_SHARED((N, F), jnp.float32),
            [pltpu.SemaphoreType.DMA] * NBUF,
            [pltpu.SemaphoreType.DMA] * NBUF,
        ],
    )
    def agg(table_hbm, src_hbm, dst_hbm, zeros_hbm, out_hbm,
            sidx_v, didx_v, rows, acc_sh, gsem, ssem):
        c = lax.axis_index("c")
        s = lax.axis_index("s")
        chunk0 = c * (EPC // CHUNK) + s * NCHUNKS
        pltpu.sync_copy(src_hbm.at[pl.ds(chunk0, NCHUNKS)], sidx_v)
        pltpu.sync_copy(dst_hbm.at[pl.ds(chunk0, NCHUNKS)], didx_v)
        _copy_tile_rows(zeros_hbm, acc_sh, s)
        plsc.subcore_barrier()

        for b in range(NBUF):
            pltpu.async_copy(table_hbm.at[sidx_v.at[b]], rows[b], gsem[b])

        def body(g, carry):
            i0 = NBUF * g
            for b in range(NBUF):
                pltpu.make_async_copy(
                    table_hbm.at[sidx_v.at[i0 + b]], rows[b], gsem[b]).wait()
                pltpu.async_copy(rows[b], acc_sh.at[didx_v.at[i0 + b]],
                                 ssem[b], add=True)
            for b in range(NBUF):
                nxt = i0 + b + NBUF

                @pl.when(nxt < NCHUNKS)
                def _():
                    pltpu.make_async_copy(
                        rows[b], acc_sh.at[didx_v.at[i0 + b]], ssem[b]).wait()
                    pltpu.async_copy(
                        table_hbm.at[sidx_v.at[nxt]], rows[b], gsem[b])
            return carry

        lax.fori_loop(0, NGRP, body, 0)
        for r in range(NGRP * NBUF, NCHUNKS):
            b = r % NBUF
            pltpu.make_async_copy(
                table_hbm.at[sidx_v.at[r]], rows[b], gsem[b]).wait()
            pltpu.async_copy(rows[b], acc_sh.at[didx_v.at[r]],
                             ssem[b], add=True)
            pltpu.make_async_copy(
                rows[b], acc_sh.at[didx_v.at[r]], ssem[b]).wait()
        for b in range(REM, NBUF):
            k = NBUF * (NGRP - 1) + b
            pltpu.make_async_copy(
                rows[b], acc_sh.at[didx_v.at[k]], ssem[b]).wait()
        plsc.subcore_barrier()
        _copy_tile_rows(acc_sh, out_hbm.at[c], s)

    return agg


@functools.cache
def _get_agg(F):
    return _make_agg(F)


_DEG_LAG = 16


def _deg_body(dst_hbm, zeros_hbm, ones_hbm, out_hbm, didx_v, ones_v, acc_sh,
              ssem):
    c = lax.axis_index("c")
    s = lax.axis_index("s")
    chunk0 = c * (EPC // CHUNK) + s * NCHUNKS
    pltpu.sync_copy(ones_hbm, ones_v)
    pltpu.sync_copy(dst_hbm.at[pl.ds(chunk0, NCHUNKS)], didx_v)
    _copy_tile_rows(zeros_hbm, acc_sh, s)
    plsc.subcore_barrier()

    def body(i, carry):
        pltpu.async_copy(ones_v, acc_sh.at[didx_v.at[i]], ssem, add=True)

        @pl.when(i >= _DEG_LAG)
        def _():
            pltpu.make_async_copy(
                ones_v, acc_sh.at[didx_v.at[i]], ssem).wait()
        return carry

    lax.fori_loop(0, NCHUNKS, body, 0)

    def drain(i, carry):
        pltpu.make_async_copy(ones_v, acc_sh.at[didx_v.at[0]], ssem).wait()
        return carry

    lax.fori_loop(0, _DEG_LAG, drain, 0)
    plsc.subcore_barrier()
    _copy_tile_rows(acc_sh, out_hbm.at[c], s)


@functools.cache
def _get_deg():
    return functools.partial(
        pl.kernel,
        out_type=jax.ShapeDtypeStruct((NC, N, DW), jnp.float32),
        mesh=_mesh(),
        compiler_params=pltpu.CompilerParams(use_tc_tiling_on_sc=False),
        scratch_types=[
            pltpu.VMEM((NCHUNKS, CHUNK), jnp.int32),
            pltpu.VMEM((CHUNK, DW), jnp.float32),
            pltpu.VMEM_SHARED((N, DW), jnp.float32),
            pltpu.SemaphoreType.DMA,
        ],
    )(_deg_body)


def _dinv_xp_body(degp_ref, x_ref, dinv_ref, xp_ref):
    dp = degp_ref[...]
    deg = dp[0, :, 0:1] + dp[1, :, 0:1] + 1.0
    dinv = lax.rsqrt(deg)
    dinv_ref[...] = dinv
    xp_ref[...] = x_ref[...] * dinv


_dinv_xp = pl.pallas_call(
    _dinv_xp_body,
    out_shape=[
        jax.ShapeDtypeStruct((N, 1), jnp.float32),
        jax.ShapeDtypeStruct((N, DIN), jnp.float32),
    ],
)


def _l1_body(s_ref, xp_ref, dinv_ref, W1_ref, b1_ref, W2_ref, out_ref):
    sp = s_ref[...]
    dinv = dinv_ref[...]
    agg = (sp[0] + sp[1] + xp_ref[...]) * dinv
    h1 = jnp.maximum(
        jnp.dot(agg, W1_ref[...], preferred_element_type=jnp.float32)
        + b1_ref[...], 0.0)
    out_ref[...] = jnp.dot(
        h1, W2_ref[...], preferred_element_type=jnp.float32) * dinv


_l1 = pl.pallas_call(
    _l1_body,
    out_shape=jax.ShapeDtypeStruct((N, 128), jnp.float32),
)


def _l2_body(s_ref, tp2_ref, dinv_ref, b2_ref, W3_ref, out_ref):
    sp = s_ref[...]
    dinv = dinv_ref[...]
    h2 = jnp.maximum((sp[0] + sp[1] + tp2_ref[...]) * dinv + b2_ref[...], 0.0)
    out_ref[...] = jnp.dot(
        h2, W3_ref[...], preferred_element_type=jnp.float32) * dinv


_l2 = pl.pallas_call(
    _l2_body,
    out_shape=jax.ShapeDtypeStruct((N, 64), jnp.float32),
)


def _l3_body(s_ref, tp3_ref, dinv_ref, b3_ref, out_ref):
    sp = s_ref[...]
    out_ref[...] = jnp.maximum(
        (sp[0] + sp[1] + tp3_ref[...]) * dinv_ref[...] + b3_ref[...], 0.0)


_l3 = pl.pallas_call(
    _l3_body,
    out_shape=jax.ShapeDtypeStruct((N, 64), jnp.float32),
)


def _final_body(h3_ref, batch_ref,
                fW1_ref, fb1_ref, fW2_ref, fb2_ref, fW3_ref, fb3_ref,
                out_ref):
    h3 = h3_ref[...]
    bid = batch_ref[...]
    neg = jnp.float32(-jnp.inf)

    NB = N // 8
    cm = h3
    for stride in (1, 2, 4):
        idsh = jnp.concatenate(
            [jnp.full((stride, 1), -1, jnp.int32), bid[:-stride]], axis=0)
        vsh = jnp.concatenate(
            [jnp.zeros((stride, 64), jnp.float32), cm[:-stride]], axis=0)
        cm = jnp.where(idsh == bid, jnp.maximum(cm, vsh), cm)
    t = cm.reshape(NB, 8, 64)[:, 7, :]
    lb = bid.reshape(NB, 8, 1)[:, 7, :]
    stride = 1
    while stride < NB:
        lbsh = jnp.concatenate(
            [jnp.full((stride, 1), -1, jnp.int32), lb[:-stride]], axis=0)
        tsh = jnp.concatenate(
            [jnp.zeros((stride, 64), jnp.float32), t[:-stride]], axis=0)
        t = jnp.where(lbsh == lb, jnp.maximum(t, tsh), t)
        stride *= 2
    scp = jnp.concatenate(
        [jnp.zeros((1, 64), jnp.float32), t[:-1]], axis=0)
    lbp = jnp.concatenate(
        [jnp.full((1, 1), -1, jnp.int32), lb[:-1]], axis=0)
    scp_r = jnp.broadcast_to(scp[:, None, :], (NB, 8, 64)).reshape(N, 64)
    lbp_r = jnp.broadcast_to(lbp[:, None, :], (NB, 8, 1)).reshape(N, 1)
    cm = jnp.where(bid == lbp_r, jnp.maximum(cm, scp_r), cm)

    nid = jnp.concatenate(
        [bid[1:], jnp.full((1, 1), -1, jnp.int32)], axis=0)
    is_last = nid != bid
    iota_g = lax.broadcasted_iota(jnp.int32, (1, G), 1)
    onehot = jnp.where((bid == iota_g) & is_last, 1.0, 0.0)
    cmz = jnp.where(is_last, cm, 0.0)
    p = lax.dot_general(onehot, cmz, (((0,), (0,)), ((), ())),
                        preferred_element_type=jnp.float32)
    cnt = lax.dot_general(onehot, jnp.ones((N, 1), jnp.float32),
                          (((0,), (0,)), ((), ())),
                          preferred_element_type=jnp.float32)
    p = jnp.where(cnt > 0.0, p, neg)
    z = jnp.maximum(
        jnp.dot(p, fW1_ref[...], preferred_element_type=jnp.float32)
        + fb1_ref[...], 0.0)
    z = jnp.maximum(
        jnp.dot(z, fW2_ref[...], preferred_element_type=jnp.float32)
        + fb2_ref[...], 0.0)
    z = jnp.dot(z, fW3_ref[...], preferred_element_type=jnp.float32) \
        + fb3_ref[...]
    zm = jnp.max(z, axis=1, keepdims=True)
    e = z - zm
    out_ref[...] = e - jnp.log(jnp.sum(jnp.exp(e), axis=1, keepdims=True))


_final = pl.pallas_call(
    _final_body,
    out_shape=jax.ShapeDtypeStruct((G, 10), jnp.float32),
)


def kernel(x, edge_index, batch,
           W1, b1, W2, b2, W3, b3,
           fW1, fb1, fW2, fb2, fW3, fb3):
    src = edge_index[0]
    dst = edge_index[1]
    dst2d = dst.reshape(E // CHUNK, CHUNK)
    src2d = lax.optimization_barrier(src).reshape(E // CHUNK, CHUNK)
    zeros128 = jnp.zeros((N, 128), jnp.float32)
    zeros64 = jnp.zeros((N, 64), jnp.float32)
    zeros16 = jnp.zeros((N, DW), jnp.float32)
    ones16 = jnp.ones((CHUNK, DW), jnp.float32)

    degp = _get_deg()(dst2d, zeros16, ones16)
    dinv, xp = _dinv_xp(degp, x.astype(jnp.float32))

    s1 = _get_agg(128)(xp, src2d, dst2d, zeros128)
    tp2 = _l1(s1, xp, dinv, W1, b1.reshape(1, -1), W2)

    s2 = _get_agg(128)(tp2, src2d, dst2d, zeros128)
    tp3 = _l2(s2, tp2, dinv, b2.reshape(1, -1), W3)

    s3 = _get_agg(64)(tp3, src2d, dst2d, zeros64)
    h3 = _l3(s3, tp3, dinv, b3.reshape(1, -1))
    out = _final(h3, batch.reshape(-1, 1),
                 fW1, fb1.reshape(1, -1), fW2, fb2.reshape(1, -1),
                 fW3, fb3.reshape(1, -1))
    return out
